# baseline (speedup 1.0000x reference)
import functools

import jax
import jax.numpy as jnp
from jax import lax
from jax.experimental import pallas as pl
from jax.experimental.pallas import tpu as pltpu
from jax.experimental.pallas import tpu_sc as plsc

_E = 64
_M = 32
_NW = 32


def _routing_body(x_ref, rw_ref, xs_ref, pos_ref, te_ref):
    xt = x_ref[...]
    T = xt.shape[0]
    logits = lax.dot_general(xt, rw_ref[...], (((1,), (1,)), ((), ())),
                             preferred_element_type=jnp.float32)
    m = jnp.max(logits, axis=1, keepdims=True)
    ex = jnp.exp(logits - m)
    s = jnp.sum(ex, axis=1, keepdims=True)
    p = ex / s
    pm = jnp.max(p, axis=1, keepdims=True)
    iota_e = lax.broadcasted_iota(jnp.int32, p.shape, 1)
    eid = jnp.min(jnp.where(p == pm, iota_e, _E), axis=1, keepdims=True)
    oh = (iota_e == eid).astype(jnp.float32)

    csum = oh
    k = 1
    while k < T:
        csum = csum + jnp.concatenate(
            [jnp.zeros((k, _E), jnp.float32), csum[:T - k]], axis=0)
        k *= 2
    rank = jnp.sum(csum * oh, axis=1, keepdims=True) - 1.0

    counts = jnp.sum(oh, axis=0, keepdims=True)
    pt = (counts.astype(jnp.int32) + (_M - 1)) // _M
    ptf = pt.astype(jnp.float32)
    iu0 = lax.broadcasted_iota(jnp.int32, (_E, _E), 0)
    iu1 = lax.broadcasted_iota(jnp.int32, (_E, _E), 1)
    tri = (iu0 <= iu1).astype(jnp.float32)
    ic = lax.dot_general(ptf, tri, (((1,), (0,)), ((), ())),
                         preferred_element_type=jnp.float32)
    po = _M * (ic - ptf)
    pot = jnp.sum(oh * po, axis=1, keepdims=True)
    pos_ref[...] = (pot + rank).astype(jnp.int32)

    ici = ic.astype(jnp.int32)
    iog = lax.broadcasted_iota(jnp.int32, (128, _E), 0)
    te = jnp.sum((iog >= ici).astype(jnp.int32), axis=1, keepdims=True)
    te_ref[...] = jnp.minimum(te, _E - 1)

    xs_ref[...] = xt * pm


def _gmm_body(te_ref, xs_ref, w_ref, ys_ref):
    ys_ref[...] = jnp.dot(xs_ref[...], w_ref[0],
                          preferred_element_type=jnp.float32)


def _sc_gather(table, idx, chunk):
    R = idx.shape[0]
    D = table.shape[1]
    per_w = R // _NW
    n_chunks = per_w // chunk
    mesh = plsc.VectorSubcoreMesh(core_axis_name="c", subcore_axis_name="s")

    @functools.partial(
        pl.kernel, mesh=mesh,
        out_type=jax.ShapeDtypeStruct((R, D), jnp.float32),
        scratch_types=[
            pltpu.VMEM((chunk,), jnp.int32),
            pltpu.VMEM((chunk, D), jnp.float32),
            pltpu.SemaphoreType.DMA,
        ])
    def k(table_hbm, idx_hbm, out_hbm, idx_v, rows_v, sem):
        wid = lax.axis_index("s") * 2 + lax.axis_index("c")
        base = wid * per_w
        for c in range(n_chunks):
            off = base + c * chunk
            pltpu.sync_copy(idx_hbm.at[pl.ds(off, chunk)], idx_v)
            pltpu.async_copy(table_hbm.at[idx_v], rows_v, sem).wait()
            pltpu.sync_copy(rows_v, out_hbm.at[pl.ds(off, chunk)])

    return k(table, idx)


def _sc_scatter_rows(rows, idx, n_out):
    R, D = rows.shape
    per_w = R // _NW
    mesh = plsc.VectorSubcoreMesh(core_axis_name="c", subcore_axis_name="s")

    @functools.partial(
        pl.kernel, mesh=mesh,
        out_type=jax.ShapeDtypeStruct((n_out, D), jnp.float32),
        scratch_types=[
            pltpu.VMEM((per_w,), jnp.int32),
            pltpu.VMEM((per_w, D), jnp.float32),
            pltpu.SemaphoreType.DMA,
        ])
    def k(rows_hbm, idx_hbm, out_hbm, idx_v, rows_v, sem):
        wid = lax.axis_index("s") * 2 + lax.axis_index("c")
        base = wid * per_w
        pltpu.sync_copy(rows_hbm.at[pl.ds(base, per_w)], rows_v)
        pltpu.sync_copy(idx_hbm.at[pl.ds(base, per_w)], idx_v)
        pltpu.async_copy(rows_v, out_hbm.at[idx_v], sem).wait()

    return k(rows, idx)


def kernel(x, router_w, expert_weights):
    B, S, H = x.shape
    E, _, D = expert_weights.shape
    T = B * S
    G = T // _M + E
    P = G * _M

    xt = x.reshape(T, H)
    xs_scaled, pos, _te = pl.pallas_call(
        _routing_body,
        out_shape=(jax.ShapeDtypeStruct((T, H), jnp.float32),
                   jax.ShapeDtypeStruct((T, 1), jnp.int32),
                   jax.ShapeDtypeStruct((128, 1), jnp.int32)),
    )(xt, router_w)
    te = _te.reshape(128)[:G]

    xs = _sc_scatter_rows(xs_scaled, pos.reshape(T), n_out=P)

    ys = pl.pallas_call(
        _gmm_body,
        grid_spec=pltpu.PrefetchScalarGridSpec(
            num_scalar_prefetch=1,
            grid=(G,),
            in_specs=[
                pl.BlockSpec((_M, H), lambda g, te_s: (g, 0)),
                pl.BlockSpec((1, H, D), lambda g, te_s: (te_s[g], 0, 0)),
            ],
            out_specs=pl.BlockSpec((_M, D), lambda g, te_s: (g, 0)),
        ),
        out_shape=jax.ShapeDtypeStruct((P, D), jnp.float32),
    )(te, xs, expert_weights)

    out = _sc_gather(ys, pos.reshape(T), chunk=64)
    return out.reshape(B, S, D)


# speedup vs baseline: 1.2395x; 1.2395x over previous
import functools

import jax
import jax.numpy as jnp
from jax import lax
from jax.experimental import pallas as pl
from jax.experimental.pallas import tpu as pltpu
from jax.experimental.pallas import tpu_sc as plsc

_E = 64
_M = 128
_NW = 32


def _routing_body(x_ref, rw_ref, g16_ref, pos_ref, te_ref):
    xt = x_ref[...]
    T = xt.shape[0]
    logits = lax.dot_general(xt, rw_ref[...], (((1,), (1,)), ((), ())),
                             preferred_element_type=jnp.float32)
    m = jnp.max(logits, axis=1, keepdims=True)
    ex = jnp.exp(logits - m)
    s = jnp.sum(ex, axis=1, keepdims=True)
    p = ex / s
    pm = jnp.max(p, axis=1, keepdims=True)
    iota_e = lax.broadcasted_iota(jnp.int32, p.shape, 1)
    eid = jnp.min(jnp.where(p == pm, iota_e, _E), axis=1, keepdims=True)
    oh = (iota_e == eid).astype(jnp.float32)

    csum = oh
    k = 1
    while k < T:
        csum = csum + jnp.concatenate(
            [jnp.zeros((k, _E), jnp.float32), csum[:T - k]], axis=0)
        k *= 2
    rank = jnp.sum(csum * oh, axis=1, keepdims=True) - 1.0

    counts = jnp.sum(oh, axis=0, keepdims=True)
    pt = (counts.astype(jnp.int32) + (_M - 1)) // _M
    ptf = pt.astype(jnp.float32)
    iu0 = lax.broadcasted_iota(jnp.int32, (_E, _E), 0)
    iu1 = lax.broadcasted_iota(jnp.int32, (_E, _E), 1)
    tri = (iu0 <= iu1).astype(jnp.float32)
    ic = lax.dot_general(ptf, tri, (((1,), (0,)), ((), ())),
                         preferred_element_type=jnp.float32)
    po = _M * (ic - ptf)
    pot = jnp.sum(oh * po, axis=1, keepdims=True)
    pos_ref[...] = (pot + rank).astype(jnp.int32)

    ici = ic.astype(jnp.int32)
    iog = lax.broadcasted_iota(jnp.int32, (128, _E), 0)
    te = jnp.sum((iog >= ici).astype(jnp.int32), axis=1, keepdims=True)
    te_ref[...] = jnp.minimum(te, _E - 1)

    g16_ref[...] = jnp.broadcast_to(pm, (T, 128))


def _gmm_body(te_ref, xs_ref, gs_ref, w_ref, ys_ref):
    ys_ref[...] = gs_ref[:, :1] * jnp.dot(xs_ref[...], w_ref[0],
                                          preferred_element_type=jnp.float32)


def _sc_gather(table, idx, chunk):
    R = idx.shape[0]
    D = table.shape[1]
    per_w = R // _NW
    n_chunks = per_w // chunk
    mesh = plsc.VectorSubcoreMesh(core_axis_name="c", subcore_axis_name="s")

    @functools.partial(
        pl.kernel, mesh=mesh,
        out_type=jax.ShapeDtypeStruct((R, D), jnp.float32),
        scratch_types=[
            pltpu.VMEM((chunk,), jnp.int32),
            pltpu.VMEM((chunk, D), jnp.float32),
            pltpu.SemaphoreType.DMA,
        ])
    def k(table_hbm, idx_hbm, out_hbm, idx_v, rows_v, sem):
        wid = lax.axis_index("s") * 2 + lax.axis_index("c")
        base = wid * per_w
        for c in range(n_chunks):
            off = base + c * chunk
            pltpu.sync_copy(idx_hbm.at[pl.ds(off, chunk)], idx_v)
            pltpu.async_copy(table_hbm.at[idx_v], rows_v, sem).wait()
            pltpu.sync_copy(rows_v, out_hbm.at[pl.ds(off, chunk)])

    return k(table, idx)


def _sc_scatter_dispatch(rows, gates, idx, n_out):
    R, D = rows.shape
    Dg = gates.shape[1]
    per_w = R // _NW
    mesh = plsc.VectorSubcoreMesh(core_axis_name="c", subcore_axis_name="s")

    @functools.partial(
        pl.kernel, mesh=mesh,
        out_type=(jax.ShapeDtypeStruct((n_out, D), jnp.float32),
                  jax.ShapeDtypeStruct((n_out, Dg), jnp.float32)),
        scratch_types=[
            pltpu.VMEM((per_w,), jnp.int32),
            pltpu.VMEM((per_w, D), jnp.float32),
            pltpu.VMEM((per_w, Dg), jnp.float32),
            pltpu.SemaphoreType.DMA,
            pltpu.SemaphoreType.DMA,
        ])
    def k(rows_hbm, g_hbm, idx_hbm, out_hbm, gs_hbm, idx_v, rows_v, g_v,
          sem_r, sem_g):
        wid = lax.axis_index("s") * 2 + lax.axis_index("c")
        base = wid * per_w
        pltpu.sync_copy(rows_hbm.at[pl.ds(base, per_w)], rows_v)
        pltpu.sync_copy(g_hbm.at[pl.ds(base, per_w)], g_v)
        pltpu.sync_copy(idx_hbm.at[pl.ds(base, per_w)], idx_v)
        cp_r = pltpu.async_copy(rows_v, out_hbm.at[idx_v], sem_r)
        cp_g = pltpu.async_copy(g_v, gs_hbm.at[idx_v], sem_g)
        cp_r.wait()
        cp_g.wait()

    return k(rows, gates, idx)


def kernel(x, router_w, expert_weights):
    B, S, H = x.shape
    E, _, D = expert_weights.shape
    T = B * S
    G = T // _M + E
    P = G * _M

    xt = x.reshape(T, H)
    g16, pos, _te = pl.pallas_call(
        _routing_body,
        out_shape=(jax.ShapeDtypeStruct((T, 128), jnp.float32),
                   jax.ShapeDtypeStruct((T, 1), jnp.int32),
                   jax.ShapeDtypeStruct((128, 1), jnp.int32)),
    )(xt, router_w)
    te = _te.reshape(128)[:G]

    xs, gs = _sc_scatter_dispatch(xt, g16, pos.reshape(T), n_out=P)

    ys = pl.pallas_call(
        _gmm_body,
        grid_spec=pltpu.PrefetchScalarGridSpec(
            num_scalar_prefetch=1,
            grid=(G,),
            in_specs=[
                pl.BlockSpec((_M, H), lambda g, te_s: (g, 0)),
                pl.BlockSpec((_M, 128), lambda g, te_s: (g, 0)),
                pl.BlockSpec((1, H, D), lambda g, te_s: (te_s[g], 0, 0)),
            ],
            out_specs=pl.BlockSpec((_M, D), lambda g, te_s: (g, 0)),
        ),
        out_shape=jax.ShapeDtypeStruct((P, D), jnp.float32),
    )(te, xs, gs, expert_weights)

    out = _sc_gather(ys, pos.reshape(T), chunk=64)
    return out.reshape(B, S, D)


# speedup vs baseline: 1.2742x; 1.0280x over previous
import functools

import jax
import jax.numpy as jnp
from jax import lax
from jax.experimental import pallas as pl
from jax.experimental.pallas import tpu as pltpu
from jax.experimental.pallas import tpu_sc as plsc

_E = 64
_M = 128
_NW = 32


def _routing_body(x_ref, rw_ref, g16_ref, pos_ref, te_ref, fb_ref, eo_ref,
                  el_ref, nu_ref):
    xt = x_ref[...]
    T = xt.shape[0]
    logits = lax.dot_general(xt, rw_ref[...], (((1,), (1,)), ((), ())),
                             preferred_element_type=jnp.float32)
    m = jnp.max(logits, axis=1, keepdims=True)
    ex = jnp.exp(logits - m)
    s = jnp.sum(ex, axis=1, keepdims=True)
    p = ex / s
    pm = jnp.max(p, axis=1, keepdims=True)
    iota_e = lax.broadcasted_iota(jnp.int32, p.shape, 1)
    eid = jnp.min(jnp.where(p == pm, iota_e, _E), axis=1, keepdims=True)
    oh = (iota_e == eid).astype(jnp.float32)

    csum = oh
    k = 1
    while k < T:
        csum = csum + jnp.concatenate(
            [jnp.zeros((k, _E), jnp.float32), csum[:T - k]], axis=0)
        k *= 2
    rank = jnp.sum(csum * oh, axis=1, keepdims=True) - 1.0

    counts = jnp.sum(oh, axis=0, keepdims=True)
    pt = (counts.astype(jnp.int32) + (_M - 1)) // _M
    ptf = pt.astype(jnp.float32)
    iu0 = lax.broadcasted_iota(jnp.int32, (_E, _E), 0)
    iu1 = lax.broadcasted_iota(jnp.int32, (_E, _E), 1)
    tri = (iu0 <= iu1).astype(jnp.float32)
    ic = lax.dot_general(ptf, tri, (((1,), (0,)), ((), ())),
                         preferred_element_type=jnp.float32)
    po = _M * (ic - ptf)
    pot = jnp.sum(oh * po, axis=1, keepdims=True)
    pos_ref[...] = (pot + rank).astype(jnp.int32)

    ici = ic.astype(jnp.int32)
    iog = lax.broadcasted_iota(jnp.int32, (128, _E), 0)
    te = jnp.sum((iog >= ici).astype(jnp.int32), axis=1, keepdims=True)
    te_ref[...] = jnp.minimum(te, _E - 1)

    pti = pt
    first_tile = ici - pti
    used = (pti > 0).astype(jnp.int32)
    fb_ref[...] = jnp.sum(((iog == first_tile) & (used == 1)).astype(jnp.int32),
                          axis=1, keepdims=True)
    eo = jnp.sum(((first_tile <= iog) & (used == 1)).astype(jnp.int32),
                 axis=1, keepdims=True) - 1
    eo_ref[...] = jnp.maximum(eo, 0)
    uo = lax.dot_general(used.astype(jnp.float32), tri, (((1,), (0,)), ((), ())),
                         preferred_element_type=jnp.float32).astype(jnp.int32)
    ind = ((iog == (uo - 1)) & (used == 1)).astype(jnp.int32)
    iota_er = lax.broadcasted_iota(jnp.int32, (128, _E), 1)
    el_ref[...] = jnp.sum(ind * iota_er, axis=1, keepdims=True)
    nu_ref[...] = jnp.sum(used, axis=1, keepdims=True)

    g16_ref[...] = jnp.broadcast_to(pm, (T, 128))


_NBUF = 4


def _gmm_body(te_s, fb_s, eo_s, el_s, nu_s, xs_ref, gs_ref, w_hbm, ys_ref,
              w_buf, sems):
    g = pl.program_id(0)
    eo = eo_s[g]
    slot = lax.rem(eo, _NBUF)
    nu = nu_s[0]

    @pl.when(g == 0)
    def _():
        for o in range(_NBUF):
            @pl.when(o < nu)
            def _():
                pltpu.make_async_copy(
                    w_hbm.at[pl.ds(el_s[o], 1)],
                    w_buf.at[pl.ds(o, 1)],
                    sems.at[o]).start()

    first = fb_s[g] == 1

    @pl.when(jnp.logical_and(first, jnp.logical_and(g > 0,
                                                    eo + _NBUF - 1 < nu)))
    def _():
        issue_ord = eo + _NBUF - 1
        islot = lax.rem(issue_ord, _NBUF)
        pltpu.make_async_copy(
            w_hbm.at[pl.ds(el_s[issue_ord], 1)],
            w_buf.at[pl.ds(islot, 1)],
            sems.at[islot]).start()

    @pl.when(first)
    def _():
        pltpu.make_async_copy(
            w_hbm.at[pl.ds(el_s[eo], 1)],
            w_buf.at[pl.ds(slot, 1)],
            sems.at[slot]).wait()

    w = w_buf[pl.ds(slot, 1), :, :]
    ys_ref[...] = gs_ref[:, :1] * jnp.dot(xs_ref[...], w[0],
                                          preferred_element_type=jnp.float32)


def _sc_gather(table, idx, chunk):
    R = idx.shape[0]
    D = table.shape[1]
    per_w = R // _NW
    n_chunks = per_w // chunk
    mesh = plsc.VectorSubcoreMesh(core_axis_name="c", subcore_axis_name="s")

    @functools.partial(
        pl.kernel, mesh=mesh,
        out_type=jax.ShapeDtypeStruct((R, D), jnp.float32),
        scratch_types=[
            pltpu.VMEM((chunk,), jnp.int32),
            pltpu.VMEM((chunk, D), jnp.float32),
            pltpu.SemaphoreType.DMA,
        ])
    def k(table_hbm, idx_hbm, out_hbm, idx_v, rows_v, sem):
        wid = lax.axis_index("s") * 2 + lax.axis_index("c")
        base = wid * per_w
        for c in range(n_chunks):
            off = base + c * chunk
            pltpu.sync_copy(idx_hbm.at[pl.ds(off, chunk)], idx_v)
            pltpu.async_copy(table_hbm.at[idx_v], rows_v, sem).wait()
            pltpu.sync_copy(rows_v, out_hbm.at[pl.ds(off, chunk)])

    return k(table, idx)


def _sc_scatter_dispatch(rows, gates, idx, n_out):
    R, D = rows.shape
    Dg = gates.shape[1]
    per_w = R // _NW
    mesh = plsc.VectorSubcoreMesh(core_axis_name="c", subcore_axis_name="s")

    @functools.partial(
        pl.kernel, mesh=mesh,
        out_type=(jax.ShapeDtypeStruct((n_out, D), jnp.float32),
                  jax.ShapeDtypeStruct((n_out, Dg), jnp.float32)),
        scratch_types=[
            pltpu.VMEM((per_w,), jnp.int32),
            pltpu.VMEM((per_w, D), jnp.float32),
            pltpu.VMEM((per_w, Dg), jnp.float32),
            pltpu.SemaphoreType.DMA,
            pltpu.SemaphoreType.DMA,
        ])
    def k(rows_hbm, g_hbm, idx_hbm, out_hbm, gs_hbm, idx_v, rows_v, g_v,
          sem_r, sem_g):
        wid = lax.axis_index("s") * 2 + lax.axis_index("c")
        base = wid * per_w
        pltpu.sync_copy(rows_hbm.at[pl.ds(base, per_w)], rows_v)
        pltpu.sync_copy(g_hbm.at[pl.ds(base, per_w)], g_v)
        pltpu.sync_copy(idx_hbm.at[pl.ds(base, per_w)], idx_v)
        cp_r = pltpu.async_copy(rows_v, out_hbm.at[idx_v], sem_r)
        cp_g = pltpu.async_copy(g_v, gs_hbm.at[idx_v], sem_g)
        cp_r.wait()
        cp_g.wait()

    return k(rows, gates, idx)


def kernel(x, router_w, expert_weights):
    B, S, H = x.shape
    E, _, D = expert_weights.shape
    T = B * S
    G = T // _M + E
    P = G * _M

    xt = x.reshape(T, H)
    g16, pos, _te, _fb, _eo, _el, _nu = pl.pallas_call(
        _routing_body,
        out_shape=(jax.ShapeDtypeStruct((T, 128), jnp.float32),
                   jax.ShapeDtypeStruct((T, 1), jnp.int32),
                   jax.ShapeDtypeStruct((128, 1), jnp.int32),
                   jax.ShapeDtypeStruct((128, 1), jnp.int32),
                   jax.ShapeDtypeStruct((128, 1), jnp.int32),
                   jax.ShapeDtypeStruct((128, 1), jnp.int32),
                   jax.ShapeDtypeStruct((1, 1), jnp.int32)),
    )(xt, router_w)
    te = _te.reshape(128)[:G]
    fb = _fb.reshape(128)[:G]
    eo = _eo.reshape(128)[:G]
    el = _el.reshape(128)[:E]
    nu = _nu.reshape(1)

    xs, gs = _sc_scatter_dispatch(xt, g16, pos.reshape(T), n_out=P)

    ys = pl.pallas_call(
        _gmm_body,
        grid_spec=pltpu.PrefetchScalarGridSpec(
            num_scalar_prefetch=5,
            grid=(G,),
            in_specs=[
                pl.BlockSpec((_M, H), lambda g, *_: (g, 0)),
                pl.BlockSpec((_M, 128), lambda g, *_: (g, 0)),
                pl.BlockSpec(memory_space=pl.ANY),
            ],
            out_specs=pl.BlockSpec((_M, D), lambda g, *_: (g, 0)),
            scratch_shapes=[
                pltpu.VMEM((_NBUF, H, D), jnp.float32),
                pltpu.SemaphoreType.DMA((_NBUF,)),
            ],
        ),
        out_shape=jax.ShapeDtypeStruct((P, D), jnp.float32),
    )(te, fb, eo, el, nu, xs, gs, expert_weights)

    out = _sc_gather(ys, pos.reshape(T), chunk=64)
    return out.reshape(B, S, D)
